# baseline (speedup 1.0000x reference)
import functools

import jax
import jax.numpy as jnp
from jax import lax
from jax.experimental import pallas as pl
from jax.experimental.pallas import tpu as pltpu
from jax.experimental.pallas import tpu_sc as plsc

_L = 16
_NC = 2
_NS = 16
_NW = _NC * _NS
_SHARD = 65536
_RW = 128
_TROWS = _NS * _SHARD // _RW


def _shift_up_one(x, lane):
    perm = jnp.minimum(lane + 1, _L - 1)
    dn = lax.GatherDimensionNumbers(
        offset_dims=(), collapsed_slice_dims=(0,), start_index_map=(0,))
    return lax.gather(x, perm[:, None], dn, slice_sizes=(1,),
                      mode=lax.GatherScatterMode.PROMISE_IN_BOUNDS)


def _make_sc_kernel(M, B, D):
    n_vregs = B // _L
    unroll = 8
    nb = B // _NW
    gb = 64
    ng = nb // gb
    hb = nb // 2
    mesh = plsc.VectorSubcoreMesh(core_axis_name="c", subcore_axis_name="s")

    @functools.partial(
        pl.kernel,
        mesh=mesh,
        compiler_params=pltpu.CompilerParams(needs_layout_passes=False),
        out_type=(
            jax.ShapeDtypeStruct((B, D), jnp.float32),
            jax.ShapeDtypeStruct((_NC, _TROWS, _RW), jnp.int32),
        ),
        scratch_types=[
            pltpu.VMEM((B,), jnp.int32),
            pltpu.VMEM((_SHARD // _RW, _RW), jnp.int32),
            pltpu.VMEM((nb,), jnp.int32),
            pltpu.VMEM((nb,), jnp.int32),
            pltpu.VMEM((2, gb, D), jnp.float32),
            pltpu.SemaphoreType.DMA,
            pltpu.SemaphoreType.DMA,
            pltpu.SemaphoreType.DMA,
            pltpu.SemaphoreType.DMA,
        ],
    )
    def sc_kernel(idx_hbm, val_hbm, out_hbm, table_hbm, idx_v, tslice,
                  rowid_v, w_v, rows_v, tsem0, tsem1, vsem0, vsem1):
        c = lax.axis_index("c")
        s = lax.axis_index("s")
        wid = s * _NC + c
        tsems = (tsem0, tsem1)
        vsems = (vsem0, vsem1)

        pltpu.sync_copy(idx_hbm, idx_v)
        lane = lax.iota(jnp.int32, _L)

        def scan_one(k):
            v = idx_v[pl.ds(k * _L, _L)]
            key = v * _L + lane
            pos = k * _L + lane
            skey, spos = plsc.sort_key_val(key, pos)
            sidx = skey >> 4
            nxt = _shift_up_one(sidx, lane)
            is_win = (lane == (_L - 1)) | (sidx != nxt)
            mine = (sidx >> 16) == s
            rel = sidx & (_SHARD - 1)
            plsc.store_scatter(tslice, [rel >> 7, rel & (_RW - 1)], spos,
                               mask=is_win & mine)

        def body(k, carry):
            for u in range(unroll):
                scan_one(k * unroll + u)
            return carry

        lax.fori_loop(0, n_vregs // unroll, body, None)

        rows_per_shard = _SHARD // _RW
        pltpu.sync_copy(
            tslice, table_hbm.at[c, pl.ds(s * rows_per_shard,
                                          rows_per_shard)])

        base = wid * nb

        def rowids(k, carry):
            v = idx_v[pl.ds(base + k * _L, _L)]
            rowid_v[pl.ds(k * _L, _L)] = v >> 7
            return carry

        lax.fori_loop(0, nb // _L, rowids, None)
        plsc.subcore_barrier()

        def start_tgather(h):
            return pltpu.async_copy(
                table_hbm.at[c].at[rowid_v.at[pl.ds(h * hb, hb)]],
                tslice.at[pl.ds(h * hb, hb)], tsems[h])

        def extract_half(h):
            for k in range(hb // _L):
                e = h * hb + k * _L
                v = idx_v[pl.ds(base + e, _L)]
                w = plsc.load_gather(tslice, [e + lane, v & (_RW - 1)])
                w_v[pl.ds(e, _L)] = w

        def start_vgather(g):
            return pltpu.async_copy(
                val_hbm.at[w_v.at[pl.ds(g * gb, gb)]],
                rows_v.at[g % 2], vsems[g % 2])

        def drain_vgather(g, vcopies):
            vcopies.pop(g).wait()
            pltpu.sync_copy(rows_v.at[g % 2],
                            out_hbm.at[pl.ds(base + g * gb, gb)])

        tcopies = [start_tgather(0), start_tgather(1)]
        vcopies = {}
        half_g = ng // 2
        tcopies[0].wait()
        extract_half(0)
        for g in range(half_g):
            vcopies[g] = start_vgather(g)
            if g > 0:
                drain_vgather(g - 1, vcopies)
        tcopies[1].wait()
        extract_half(1)
        for g in range(half_g, ng):
            vcopies[g] = start_vgather(g)
            drain_vgather(g - 1, vcopies)
        drain_vgather(ng - 1, vcopies)

    return sc_kernel


def kernel(mem, idx, val):
    del mem
    M = 1000000
    B, D = val.shape
    out, _table = _make_sc_kernel(M, B, D)(idx, val)
    return out


# speedup vs baseline: 1.0264x; 1.0264x over previous
import functools

import jax
import jax.numpy as jnp
from jax import lax
from jax.experimental import pallas as pl
from jax.experimental.pallas import tpu as pltpu
from jax.experimental.pallas import tpu_sc as plsc

_L = 16
_NC = 2
_NS = 16
_NW = _NC * _NS
_SHARD = 65536
_RW = 128
_TROWS = _NS * _SHARD // _RW


def _shift_up_one(x, lane):
    perm = jnp.minimum(lane + 1, _L - 1)
    dn = lax.GatherDimensionNumbers(
        offset_dims=(), collapsed_slice_dims=(0,), start_index_map=(0,))
    return lax.gather(x, perm[:, None], dn, slice_sizes=(1,),
                      mode=lax.GatherScatterMode.PROMISE_IN_BOUNDS)


def _make_sc_kernel(M, B, D):
    n_vregs = B // _L
    unroll = 8
    nb = B // _NW
    gb = 64
    ng = nb // gb
    hb = nb // 2
    mesh = plsc.VectorSubcoreMesh(core_axis_name="c", subcore_axis_name="s")

    @functools.partial(
        pl.kernel,
        mesh=mesh,
        compiler_params=pltpu.CompilerParams(needs_layout_passes=False),
        out_type=(
            jax.ShapeDtypeStruct((B, D), jnp.float32),
            jax.ShapeDtypeStruct((_NC, _TROWS, _RW), jnp.int32),
        ),
        scratch_types=[
            pltpu.VMEM((B,), jnp.int32),
            pltpu.VMEM((_SHARD // _RW, _RW), jnp.int32),
            pltpu.VMEM((nb,), jnp.int32),
            pltpu.VMEM((nb,), jnp.int32),
            pltpu.VMEM((4, gb, D), jnp.float32),
            pltpu.SemaphoreType.DMA,
            pltpu.SemaphoreType.DMA,
            pltpu.SemaphoreType.DMA,
            pltpu.SemaphoreType.DMA,
        ],
    )
    def sc_kernel(idx_hbm, val_hbm, out_hbm, table_hbm, idx_v, tslice,
                  rowid_v, w_v, rows_v, tsem0, tsem1, vsem0, vsem1):
        c = lax.axis_index("c")
        s = lax.axis_index("s")
        wid = s * _NC + c
        tsems = (tsem0, tsem1)
        vsems = (vsem0, vsem1)

        pltpu.sync_copy(idx_hbm, idx_v)
        lane = lax.iota(jnp.int32, _L)

        def scan_one(k):
            v = idx_v[pl.ds(k * _L, _L)]
            key = v * _L + lane
            pos = k * _L + lane
            skey, spos = plsc.sort_key_val(key, pos)
            sidx = skey >> 4
            nxt = _shift_up_one(sidx, lane)
            is_win = (lane == (_L - 1)) | (sidx != nxt)
            mine = (sidx >> 16) == s
            rel = sidx & (_SHARD - 1)
            plsc.store_scatter(tslice, [rel >> 7, rel & (_RW - 1)], spos,
                               mask=is_win & mine)

        def body(k, carry):
            for u in range(unroll):
                scan_one(k * unroll + u)
            return carry

        lax.fori_loop(0, n_vregs // unroll, body, None)

        rows_per_shard = _SHARD // _RW
        pltpu.sync_copy(
            tslice, table_hbm.at[c, pl.ds(s * rows_per_shard,
                                          rows_per_shard)])

        base = wid * nb

        def rowids(k, carry):
            v = idx_v[pl.ds(base + k * _L, _L)]
            rowid_v[pl.ds(k * _L, _L)] = v >> 7
            return carry

        lax.fori_loop(0, nb // _L, rowids, None)
        plsc.subcore_barrier()

        def start_tgather(h):
            return pltpu.async_copy(
                table_hbm.at[c].at[rowid_v.at[pl.ds(h * hb, hb)]],
                tslice.at[pl.ds(h * hb, hb)], tsems[h])

        def extract_half(h):
            for k in range(hb // _L):
                e = h * hb + k * _L
                v = idx_v[pl.ds(base + e, _L)]
                w = plsc.load_gather(tslice, [e + lane, v & (_RW - 1)])
                w_v[pl.ds(e, _L)] = w

        vdepth = 4
        all_vsems = (vsems[0], vsems[1], tsems[0], tsems[1])

        def start_vgather(g):
            return pltpu.async_copy(
                val_hbm.at[w_v.at[pl.ds(g * gb, gb)]],
                rows_v.at[g % vdepth], all_vsems[g % vdepth])

        def drain_vgather(g, vcopies):
            vcopies.pop(g).wait()
            pltpu.sync_copy(rows_v.at[g % vdepth],
                            out_hbm.at[pl.ds(base + g * gb, gb)])

        tcopies = [start_tgather(0), start_tgather(1)]
        vcopies = {}
        tcopies[0].wait()
        extract_half(0)
        for g in range(3):
            vcopies[g] = start_vgather(g)
        tcopies[1].wait()
        extract_half(1)
        for g in range(3, ng):
            vcopies[g] = start_vgather(g)
            drain_vgather(g - 3, vcopies)
        for g in range(ng - 3, ng):
            drain_vgather(g, vcopies)

    return sc_kernel


def kernel(mem, idx, val):
    del mem
    M = 1000000
    B, D = val.shape
    out, _table = _make_sc_kernel(M, B, D)(idx, val)
    return out


# speedup vs baseline: 1.2110x; 1.1798x over previous
import functools

import jax
import jax.numpy as jnp
from jax import lax
from jax.experimental import pallas as pl
from jax.experimental.pallas import tpu as pltpu
from jax.experimental.pallas import tpu_sc as plsc

_L = 16
_NC = 2
_NS = 16
_NW = _NC * _NS
_SHARD = 65536
_RW = 128
_TROWS = _NS * _SHARD // _RW
_SENT = 1 << 20


def _shift_up_one(x, lane):
    perm = jnp.minimum(lane + 1, _L - 1)
    dn = lax.GatherDimensionNumbers(
        offset_dims=(), collapsed_slice_dims=(0,), start_index_map=(0,))
    return lax.gather(x, perm[:, None], dn, slice_sizes=(1,),
                      mode=lax.GatherScatterMode.PROMISE_IN_BOUNDS)


def _make_sc_kernel(M, B, D):
    n_vregs = B // _L
    unroll = 8
    nb = B // _NW
    gb = 64
    ng = nb // gb
    hb = nb // 2
    mesh = plsc.VectorSubcoreMesh(core_axis_name="c", subcore_axis_name="s")

    @functools.partial(
        pl.kernel,
        mesh=mesh,
        compiler_params=pltpu.CompilerParams(needs_layout_passes=False),
        out_type=(
            jax.ShapeDtypeStruct((B, D), jnp.float32),
            jax.ShapeDtypeStruct((_NC, _TROWS, _RW), jnp.int32),
            jax.ShapeDtypeStruct((_NC, B), jnp.int32),
        ),
        scratch_types=[
            pltpu.VMEM((B,), jnp.int32),
            pltpu.VMEM((_SHARD // _RW, _RW), jnp.int32),
            pltpu.VMEM((B,), jnp.int32),
            pltpu.VMEM((nb,), jnp.int32),
            pltpu.VMEM((nb,), jnp.int32),
            pltpu.VMEM((3, gb, D), jnp.float32),
            pltpu.SemaphoreType.DMA,
            pltpu.SemaphoreType.DMA,
            pltpu.SemaphoreType.DMA,
            pltpu.SemaphoreType.DMA,
        ],
    )
    def sc_kernel(idx_hbm, val_hbm, out_hbm, table_hbm, midx_hbm, idx_v,
                  tslice, midx_v, rowid_v, w_v, rows_v, tsem0, tsem1,
                  vsem0, vsem1):
        c = lax.axis_index("c")
        s = lax.axis_index("s")
        wid = s * _NC + c
        tsems = (tsem0, tsem1)
        vsems = (vsem0, vsem1)

        pltpu.sync_copy(idx_hbm, idx_v)
        lane = lax.iota(jnp.int32, _L)
        sent_v = lane * 0 + _SENT

        seg = n_vregs // _NS

        def prep_one(kp, carry):
            off = (s * seg + kp) * _L
            v = idx_v[pl.ds(off, _L)]
            key = v * _L + lane
            skey, _ = plsc.sort_key_val(key, lane)
            sidx = skey >> 4
            nxt = _shift_up_one(sidx, lane)
            lose = (sidx == nxt) & (lane != (_L - 1))
            midx_v[pl.ds(off, _L)] = v
            plsc.store_scatter(midx_v, [off + (skey & (_L - 1))], sent_v,
                               mask=lose)
            return carry

        lax.fori_loop(0, seg, prep_one, None)
        pltpu.sync_copy(midx_v.at[pl.ds(s * seg * _L, seg * _L)],
                        midx_hbm.at[c, pl.ds(s * seg * _L, seg * _L)])
        plsc.subcore_barrier()
        pltpu.sync_copy(midx_hbm.at[c], midx_v)

        def scan_one(k):
            v = midx_v[pl.ds(k * _L, _L)]
            mine = (v >> 16) == s
            rel = v & (_SHARD - 1)
            pos = k * _L + lane
            plsc.store_scatter(tslice, [rel >> 7, rel & (_RW - 1)], pos,
                               mask=mine)

        def body(k, carry):
            for u in range(unroll):
                scan_one(k * unroll + u)
            return carry

        lax.fori_loop(0, n_vregs // unroll, body, None)

        rows_per_shard = _SHARD // _RW
        pltpu.sync_copy(
            tslice, table_hbm.at[c, pl.ds(s * rows_per_shard,
                                          rows_per_shard)])

        base = wid * nb

        def rowids(k, carry):
            v = idx_v[pl.ds(base + k * _L, _L)]
            rowid_v[pl.ds(k * _L, _L)] = v >> 7
            return carry

        lax.fori_loop(0, nb // _L, rowids, None)
        plsc.subcore_barrier()

        def start_tgather(h):
            return pltpu.async_copy(
                table_hbm.at[c].at[rowid_v.at[pl.ds(h * hb, hb)]],
                tslice.at[pl.ds(h * hb, hb)], tsems[h])

        def extract_half(h):
            for k in range(hb // _L):
                e = h * hb + k * _L
                v = idx_v[pl.ds(base + e, _L)]
                w = plsc.load_gather(tslice, [e + lane, v & (_RW - 1)])
                w_v[pl.ds(e, _L)] = w

        vdepth = 3
        all_vsems = (vsems[0], vsems[1], tsems[0])

        def start_vgather(g):
            return pltpu.async_copy(
                val_hbm.at[w_v.at[pl.ds(g * gb, gb)]],
                rows_v.at[g % vdepth], all_vsems[g % vdepth])

        def drain_vgather(g, vcopies):
            vcopies.pop(g).wait()
            pltpu.sync_copy(rows_v.at[g % vdepth],
                            out_hbm.at[pl.ds(base + g * gb, gb)])

        tcopies = [start_tgather(0), start_tgather(1)]
        vcopies = {}
        tcopies[0].wait()
        extract_half(0)
        for g in range(2):
            vcopies[g] = start_vgather(g)
        tcopies[1].wait()
        extract_half(1)
        for g in range(2, ng):
            vcopies[g] = start_vgather(g)
            drain_vgather(g - 2, vcopies)
        for g in range(ng - 2, ng):
            drain_vgather(g, vcopies)

    return sc_kernel


def kernel(mem, idx, val):
    del mem
    M = 1000000
    B, D = val.shape
    out, _table, _midx = _make_sc_kernel(M, B, D)(idx, val)
    return out


# speedup vs baseline: 1.2930x; 1.0677x over previous
import functools

import jax
import jax.numpy as jnp
from jax import lax
from jax.experimental import pallas as pl
from jax.experimental.pallas import tpu as pltpu
from jax.experimental.pallas import tpu_sc as plsc

_L = 16
_NC = 2
_NS = 16
_NW = _NC * _NS
_SHARD = 65536
_RW = 16
_RWS = 4
_TROWS = _NS * _SHARD // _RW
_SENT = 1 << 20


def _shift_up_one(x, lane):
    perm = jnp.minimum(lane + 1, _L - 1)
    dn = lax.GatherDimensionNumbers(
        offset_dims=(), collapsed_slice_dims=(0,), start_index_map=(0,))
    return lax.gather(x, perm[:, None], dn, slice_sizes=(1,),
                      mode=lax.GatherScatterMode.PROMISE_IN_BOUNDS)


def _make_sc_kernel(M, B, D):
    n_vregs = B // _L
    unroll = 8
    nb = B // _NW
    gb = 64
    ng = nb // gb
    hb = nb // 2
    mesh = plsc.VectorSubcoreMesh(core_axis_name="c", subcore_axis_name="s")

    @functools.partial(
        pl.kernel,
        mesh=mesh,
        compiler_params=pltpu.CompilerParams(needs_layout_passes=False, use_tc_tiling_on_sc=False),
        out_type=(
            jax.ShapeDtypeStruct((B, D), jnp.float32),
            jax.ShapeDtypeStruct((_NC, _TROWS, _RW), jnp.int32),
            jax.ShapeDtypeStruct((_NC, B), jnp.int32),
        ),
        scratch_types=[
            pltpu.VMEM((B,), jnp.int32),
            pltpu.VMEM((_SHARD // _RW, _RW), jnp.int32),
            pltpu.VMEM((B,), jnp.int32),
            pltpu.VMEM((nb,), jnp.int32),
            pltpu.VMEM((nb,), jnp.int32),
            pltpu.VMEM((3, gb, D), jnp.float32),
            pltpu.SemaphoreType.DMA,
            pltpu.SemaphoreType.DMA,
            pltpu.SemaphoreType.DMA,
            pltpu.SemaphoreType.DMA,
        ],
    )
    def sc_kernel(idx_hbm, val_hbm, out_hbm, table_hbm, midx_hbm, idx_v,
                  tslice, midx_v, rowid_v, w_v, rows_v, tsem0, tsem1,
                  vsem0, vsem1):
        c = lax.axis_index("c")
        s = lax.axis_index("s")
        wid = s * _NC + c
        tsems = (tsem0, tsem1)
        vsems = (vsem0, vsem1)

        pltpu.sync_copy(idx_hbm, idx_v)
        lane = lax.iota(jnp.int32, _L)
        sent_v = lane * 0 + _SENT

        seg = n_vregs // _NS

        def prep_one(kp, carry):
            off = (s * seg + kp) * _L
            v = idx_v[pl.ds(off, _L)]
            key = v * _L + lane
            skey, _ = plsc.sort_key_val(key, lane)
            sidx = skey >> 4
            nxt = _shift_up_one(sidx, lane)
            lose = (sidx == nxt) & (lane != (_L - 1))
            midx_v[pl.ds(off, _L)] = v
            plsc.store_scatter(midx_v, [off + (skey & (_L - 1))], sent_v,
                               mask=lose)
            return carry

        lax.fori_loop(0, seg, prep_one, None)
        pltpu.sync_copy(midx_v.at[pl.ds(s * seg * _L, seg * _L)],
                        midx_hbm.at[c, pl.ds(s * seg * _L, seg * _L)])
        plsc.subcore_barrier()
        pltpu.sync_copy(midx_hbm.at[c], midx_v)

        def scan_one(k):
            v = midx_v[pl.ds(k * _L, _L)]
            mine = (v >> 16) == s
            rel = v & (_SHARD - 1)
            pos = k * _L + lane
            plsc.store_scatter(tslice, [rel >> _RWS, rel & (_RW - 1)], pos,
                               mask=mine)

        def body(k, carry):
            for u in range(unroll):
                scan_one(k * unroll + u)
            return carry

        lax.fori_loop(0, n_vregs // unroll, body, None)

        rows_per_shard = _SHARD // _RW
        pltpu.sync_copy(
            tslice, table_hbm.at[c, pl.ds(s * rows_per_shard,
                                          rows_per_shard)])

        base = wid * nb

        def rowids(k, carry):
            v = idx_v[pl.ds(base + k * _L, _L)]
            rowid_v[pl.ds(k * _L, _L)] = v >> _RWS
            return carry

        lax.fori_loop(0, nb // _L, rowids, None)
        plsc.subcore_barrier()

        def start_tgather(h):
            return pltpu.async_copy(
                table_hbm.at[c].at[rowid_v.at[pl.ds(h * hb, hb)]],
                tslice.at[pl.ds(h * hb, hb)], tsems[h])

        def extract_half(h):
            for k in range(hb // _L):
                e = h * hb + k * _L
                v = idx_v[pl.ds(base + e, _L)]
                w = plsc.load_gather(tslice, [e + lane, v & (_RW - 1)])
                w_v[pl.ds(e, _L)] = w

        vdepth = 3
        all_vsems = (vsems[0], vsems[1], tsems[0])

        def start_vgather(g):
            return pltpu.async_copy(
                val_hbm.at[w_v.at[pl.ds(g * gb, gb)]],
                rows_v.at[g % vdepth], all_vsems[g % vdepth])

        def drain_vgather(g, vcopies):
            vcopies.pop(g).wait()
            pltpu.sync_copy(rows_v.at[g % vdepth],
                            out_hbm.at[pl.ds(base + g * gb, gb)])

        tcopies = [start_tgather(0), start_tgather(1)]
        vcopies = {}
        tcopies[0].wait()
        extract_half(0)
        for g in range(2):
            vcopies[g] = start_vgather(g)
        tcopies[1].wait()
        extract_half(1)
        for g in range(2, ng):
            vcopies[g] = start_vgather(g)
            drain_vgather(g - 2, vcopies)
        for g in range(ng - 2, ng):
            drain_vgather(g, vcopies)

    return sc_kernel


def kernel(mem, idx, val):
    del mem
    M = 1000000
    B, D = val.shape
    out, _table, _midx = _make_sc_kernel(M, B, D)(idx, val)
    return out


# speedup vs baseline: 1.3587x; 1.0509x over previous
import functools

import jax
import jax.numpy as jnp
from jax import lax
from jax.experimental import pallas as pl
from jax.experimental.pallas import tpu as pltpu
from jax.experimental.pallas import tpu_sc as plsc

_L = 16
_NC = 2
_NS = 16
_NW = _NC * _NS
_SHARD = 65536
_RW = 16
_RWS = 4
_TROWS = _NS * _SHARD // _RW
_SENT = 1 << 20


def _shift_up_one(x, lane):
    perm = jnp.minimum(lane + 1, _L - 1)
    dn = lax.GatherDimensionNumbers(
        offset_dims=(), collapsed_slice_dims=(0,), start_index_map=(0,))
    return lax.gather(x, perm[:, None], dn, slice_sizes=(1,),
                      mode=lax.GatherScatterMode.PROMISE_IN_BOUNDS)


def _make_sc_kernel(M, B, D):
    n_vregs = B // _L
    unroll = 8
    nb = B // _NW
    gb = 64
    ng = nb // gb
    hb = nb // 2
    mesh = plsc.VectorSubcoreMesh(core_axis_name="c", subcore_axis_name="s")

    @functools.partial(
        pl.kernel,
        mesh=mesh,
        compiler_params=pltpu.CompilerParams(needs_layout_passes=False, use_tc_tiling_on_sc=False),
        out_type=(
            jax.ShapeDtypeStruct((B, D), jnp.float32),
            jax.ShapeDtypeStruct((_NC, _TROWS, _RW), jnp.int32),
            jax.ShapeDtypeStruct((_NC, B), jnp.int32),
        ),
        scratch_types=[
            pltpu.VMEM((B // _NS,), jnp.int32),
            pltpu.VMEM((B // _NS,), jnp.int32),
            pltpu.VMEM((B // _NW,), jnp.int32),
            pltpu.VMEM((_SHARD // _RW, _RW), jnp.int32),
            pltpu.VMEM((B,), jnp.int32),
            pltpu.VMEM((nb,), jnp.int32),
            pltpu.VMEM((nb,), jnp.int32),
            pltpu.VMEM((3, gb, D), jnp.float32),
            pltpu.SemaphoreType.DMA,
            pltpu.SemaphoreType.DMA,
            pltpu.SemaphoreType.DMA,
            pltpu.SemaphoreType.DMA,
        ],
    )
    def sc_kernel(idx_hbm, val_hbm, out_hbm, table_hbm, midx_hbm, pidx_v,
                  pmidx_v, bidx_v, tslice, midx_v, rowid_v, w_v, rows_v,
                  tsem0, tsem1, vsem0, vsem1):
        c = lax.axis_index("c")
        s = lax.axis_index("s")
        wid = s * _NC + c
        tsems = (tsem0, tsem1)
        vsems = (vsem0, vsem1)

        seg_e = B // _NS
        base = wid * nb
        pltpu.sync_copy(idx_hbm.at[pl.ds(s * seg_e, seg_e)], pidx_v)
        pltpu.sync_copy(idx_hbm.at[pl.ds(base, nb)], bidx_v)
        lane = lax.iota(jnp.int32, _L)
        sent_v = lane * 0 + _SENT

        seg = n_vregs // _NS

        def prep_one(kp, carry):
            off = kp * _L
            v = pidx_v[pl.ds(off, _L)]
            key = v * _L + lane
            skey, _ = plsc.sort_key_val(key, lane)
            sidx = skey >> 4
            nxt = _shift_up_one(sidx, lane)
            lose = (sidx == nxt) & (lane != (_L - 1))
            pmidx_v[pl.ds(off, _L)] = v
            plsc.store_scatter(pmidx_v, [off + (skey & (_L - 1))], sent_v,
                               mask=lose)
            return carry

        lax.fori_loop(0, seg, prep_one, None)
        pltpu.sync_copy(pmidx_v, midx_hbm.at[c, pl.ds(s * seg_e, seg_e)])
        plsc.subcore_barrier()
        pltpu.sync_copy(midx_hbm.at[c], midx_v)

        def scan_one(k):
            v = midx_v[pl.ds(k * _L, _L)]
            mine = (v >> 16) == s
            rel = v & (_SHARD - 1)
            pos = k * _L + lane
            plsc.store_scatter(tslice, [rel >> _RWS, rel & (_RW - 1)], pos,
                               mask=mine)

        def body(k, carry):
            for u in range(unroll):
                scan_one(k * unroll + u)
            return carry

        lax.fori_loop(0, n_vregs // unroll, body, None)

        rows_per_shard = _SHARD // _RW
        pltpu.sync_copy(
            tslice, table_hbm.at[c, pl.ds(s * rows_per_shard,
                                          rows_per_shard)])

        def rowids(k, carry):
            v = bidx_v[pl.ds(k * _L, _L)]
            rowid_v[pl.ds(k * _L, _L)] = v >> _RWS
            return carry

        lax.fori_loop(0, nb // _L, rowids, None)
        plsc.subcore_barrier()

        def start_tgather(h):
            return pltpu.async_copy(
                table_hbm.at[c].at[rowid_v.at[pl.ds(h * hb, hb)]],
                tslice.at[pl.ds(h * hb, hb)], tsems[h])

        def extract_half(h):
            for k in range(hb // _L):
                e = h * hb + k * _L
                v = bidx_v[pl.ds(e, _L)]
                w = plsc.load_gather(tslice, [e + lane, v & (_RW - 1)])
                w_v[pl.ds(e, _L)] = w

        vdepth = 3
        all_vsems = (vsems[0], vsems[1], tsems[0])

        def start_vgather(g):
            return pltpu.async_copy(
                val_hbm.at[w_v.at[pl.ds(g * gb, gb)]],
                rows_v.at[g % vdepth], all_vsems[g % vdepth])

        def drain_vgather(g, vcopies):
            vcopies.pop(g).wait()
            pltpu.sync_copy(rows_v.at[g % vdepth],
                            out_hbm.at[pl.ds(base + g * gb, gb)])

        tcopies = [start_tgather(0), start_tgather(1)]
        vcopies = {}
        tcopies[0].wait()
        extract_half(0)
        for g in range(2):
            vcopies[g] = start_vgather(g)
        tcopies[1].wait()
        extract_half(1)
        for g in range(2, ng):
            vcopies[g] = start_vgather(g)
            drain_vgather(g - 2, vcopies)
        for g in range(ng - 2, ng):
            drain_vgather(g, vcopies)

    return sc_kernel


def kernel(mem, idx, val):
    del mem
    M = 1000000
    B, D = val.shape
    out, _table, _midx = _make_sc_kernel(M, B, D)(idx, val)
    return out
